# initial kernel scaffold (unmeasured)
import jax
import jax.numpy as jnp
from jax import lax
from jax.experimental import pallas as pl
from jax.experimental.pallas import tpu as pltpu

N_DEV = 4
N_EXPERTS = 32
N_LOCAL = N_EXPERTS // N_DEV
N_TOK = 2048
D_IN = 512
D_OUT = 1024
CHUNK = N_TOK // N_DEV
N_HOP = N_DEV - 1


def kernel(x, router_W, route_idx, expert_W):
    my = lax.axis_index("i")

    scores = x @ router_W
    scores = scores - jnp.max(scores, axis=-1, keepdims=True)
    probs = jnp.exp(scores)
    probs = probs / jnp.sum(probs, axis=-1, keepdims=True)

    e0 = route_idx[:, 0]
    e1 = route_idx[:, 1]
    all_ids = jnp.arange(N_EXPERTS, dtype=jnp.int32)
    g0 = jnp.sum(probs * (e0[:, None] == all_ids[None, :]), axis=1)
    g1 = jnp.sum(probs * (e1[:, None] == all_ids[None, :]), axis=1)
    gs = g0 + g1
    w0 = g0 / gs
    w1 = g1 / gs

    local_ids = (my * N_LOCAL + jnp.arange(N_LOCAL, dtype=jnp.int32))
    gates = (
        w0[:, None] * (e0[:, None] == local_ids[None, :])
        + w1[:, None] * (e1[:, None] == local_ids[None, :])
    ).astype(jnp.float32)

    def body(
        x_ref,
        g_ref,
        w_ref,
        out_ref,
        rs_buf,
        rs_send_sems,
        rs_recv_sems,
        ag_send_sems,
        ag_recv_sems,
    ):
        my_pos = lax.axis_index("i")
        left = (my_pos - 1) % N_DEV
        right = (my_pos + 1) % N_DEV

        barrier_sem = pltpu.get_barrier_semaphore()
        for nbr in [left, right]:
            pl.semaphore_signal(
                barrier_sem,
                inc=1,
                device_id=(nbr,),
                device_id_type=pl.DeviceIdType.MESH,
            )
        pl.semaphore_wait(barrier_sem, 2)

        acc = jnp.zeros((N_TOK, D_OUT), jnp.float32)
        for e in range(N_LOCAL):
            xg = x_ref[:, :] * g_ref[:, e : e + 1]
            acc = acc + jnp.dot(
                xg, w_ref[e], preferred_element_type=jnp.float32
            )
        out_ref[:, :] = acc

        for h in range(N_HOP):
            c_send = (my_pos - h) % N_DEV
            c_recv = (my_pos - h - 1) % N_DEV
            rdma = pltpu.make_async_remote_copy(
                src_ref=out_ref.at[pl.ds(c_send * CHUNK, CHUNK), :],
                dst_ref=rs_buf.at[h],
                send_sem=rs_send_sems.at[h],
                recv_sem=rs_recv_sems.at[h],
                device_id=(right,),
                device_id_type=pl.DeviceIdType.MESH,
            )
            rdma.start()
            rdma.wait()
            out_ref[pl.ds(c_recv * CHUNK, CHUNK), :] = (
                out_ref[pl.ds(c_recv * CHUNK, CHUNK), :] + rs_buf[h]
            )

        for s in range(N_HOP):
            c_send = (my_pos + 1 - s) % N_DEV
            rdma = pltpu.make_async_remote_copy(
                src_ref=out_ref.at[pl.ds(c_send * CHUNK, CHUNK), :],
                dst_ref=out_ref.at[pl.ds(c_send * CHUNK, CHUNK), :],
                send_sem=ag_send_sems.at[s],
                recv_sem=ag_recv_sems.at[s],
                device_id=(right,),
                device_id_type=pl.DeviceIdType.MESH,
            )
            rdma.start()
            rdma.wait()

    return pl.pallas_call(
        body,
        out_shape=jax.ShapeDtypeStruct((N_TOK, D_OUT), jnp.float32),
        in_specs=[
            pl.BlockSpec(memory_space=pltpu.VMEM),
            pl.BlockSpec(memory_space=pltpu.VMEM),
            pl.BlockSpec(memory_space=pltpu.VMEM),
        ],
        out_specs=pl.BlockSpec(memory_space=pltpu.VMEM),
        scratch_shapes=[
            pltpu.VMEM((N_HOP, CHUNK, D_OUT), jnp.float32),
            pltpu.SemaphoreType.DMA((N_HOP,)),
            pltpu.SemaphoreType.DMA((N_HOP,)),
            pltpu.SemaphoreType.DMA((N_HOP,)),
            pltpu.SemaphoreType.DMA((N_HOP,)),
        ],
        compiler_params=pltpu.CompilerParams(collective_id=0),
    )(x, gates, expert_W)


# baseline (device time: 190805 ns/iter reference)
import jax
import jax.numpy as jnp
from jax import lax
from jax.experimental import pallas as pl
from jax.experimental.pallas import tpu as pltpu

N_DEV = 4
N_EXPERTS = 32
N_LOCAL = N_EXPERTS // N_DEV
N_TOK = 2048
D_IN = 512
D_OUT = 1024
CHUNK = N_TOK // N_DEV
N_HOP = N_DEV - 1


def kernel(x, router_W, route_idx, expert_W):
    my = lax.axis_index("i")

    scores = x @ router_W
    scores = scores - jnp.max(scores, axis=-1, keepdims=True)
    probs = jnp.exp(scores)
    probs = probs / jnp.sum(probs, axis=-1, keepdims=True)

    e0 = route_idx[:, 0]
    e1 = route_idx[:, 1]
    all_ids = jnp.arange(N_EXPERTS, dtype=jnp.int32)
    g0 = jnp.sum(probs * (e0[:, None] == all_ids[None, :]), axis=1)
    g1 = jnp.sum(probs * (e1[:, None] == all_ids[None, :]), axis=1)
    gs = g0 + g1
    w0 = g0 / gs
    w1 = g1 / gs

    local_ids = (my * N_LOCAL + jnp.arange(N_LOCAL, dtype=jnp.int32))
    gates = (
        w0[:, None] * (e0[:, None] == local_ids[None, :])
        + w1[:, None] * (e1[:, None] == local_ids[None, :])
    ).astype(jnp.float32)

    def body(
        x_ref,
        g_ref,
        w_ref,
        out_ref,
        rs_buf,
        rs_send_sems,
        rs_recv_sems,
        ag_send_sems,
        ag_recv_sems,
    ):
        my_pos = lax.axis_index("i")
        left = (my_pos - 1) % N_DEV
        right = (my_pos + 1) % N_DEV

        barrier_sem = pltpu.get_barrier_semaphore()
        for nbr in [left, right]:
            pl.semaphore_signal(
                barrier_sem,
                inc=1,
                device_id=(nbr,),
                device_id_type=pl.DeviceIdType.MESH,
            )
        pl.semaphore_wait(barrier_sem, 2)

        acc = jnp.zeros((N_TOK, D_OUT), jnp.float32)
        for e in range(N_LOCAL):
            xg = x_ref[:, :] * g_ref[:, e : e + 1]
            acc = acc + jnp.dot(
                xg, w_ref[e], preferred_element_type=jnp.float32
            )
        out_ref[:, :] = acc

        for h in range(N_HOP):
            c_send = (my_pos - h) % N_DEV
            c_recv = (my_pos - h - 1) % N_DEV
            rdma = pltpu.make_async_remote_copy(
                src_ref=out_ref.at[pl.ds(c_send * CHUNK, CHUNK), :],
                dst_ref=rs_buf.at[h],
                send_sem=rs_send_sems.at[h],
                recv_sem=rs_recv_sems.at[h],
                device_id=(right,),
                device_id_type=pl.DeviceIdType.MESH,
            )
            rdma.start()
            rdma.wait()
            out_ref[pl.ds(c_recv * CHUNK, CHUNK), :] = (
                out_ref[pl.ds(c_recv * CHUNK, CHUNK), :] + rs_buf[h]
            )

        for s in range(N_HOP):
            c_send = (my_pos + 1 - s) % N_DEV
            rdma = pltpu.make_async_remote_copy(
                src_ref=out_ref.at[pl.ds(c_send * CHUNK, CHUNK), :],
                dst_ref=out_ref.at[pl.ds(c_send * CHUNK, CHUNK), :],
                send_sem=ag_send_sems.at[s],
                recv_sem=ag_recv_sems.at[s],
                device_id=(right,),
                device_id_type=pl.DeviceIdType.MESH,
            )
            rdma.start()
            rdma.wait()

    return pl.pallas_call(
        body,
        out_shape=jax.ShapeDtypeStruct((N_TOK, D_OUT), jnp.float32),
        in_specs=[
            pl.BlockSpec(memory_space=pltpu.VMEM),
            pl.BlockSpec(memory_space=pltpu.VMEM),
            pl.BlockSpec(memory_space=pltpu.VMEM),
        ],
        out_specs=pl.BlockSpec(memory_space=pltpu.VMEM),
        scratch_shapes=[
            pltpu.VMEM((N_HOP, CHUNK, D_OUT), jnp.float32),
            pltpu.SemaphoreType.DMA((N_HOP,)),
            pltpu.SemaphoreType.DMA((N_HOP,)),
            pltpu.SemaphoreType.DMA((N_HOP,)),
            pltpu.SemaphoreType.DMA((N_HOP,)),
        ],
        compiler_params=pltpu.CompilerParams(
            collective_id=0, vmem_limit_bytes=100 * 1024 * 1024
        ),
    )(x, gates, expert_W)


# device time: 76333 ns/iter; 2.4996x vs baseline; 2.4996x over previous
import jax
import jax.numpy as jnp
from jax import lax
from jax.experimental import pallas as pl
from jax.experimental.pallas import tpu as pltpu

N_DEV = 4
N_EXPERTS = 32
N_LOCAL = N_EXPERTS // N_DEV
N_TOK = 2048
D_IN = 512
D_OUT = 1024
CHUNK = N_TOK // N_DEV
HALF = D_OUT // 2
N_HOP = N_DEV - 1


def kernel(x, router_W, route_idx, expert_W):
    my = lax.axis_index("i")

    scores = x @ router_W
    scores = scores - jnp.max(scores, axis=-1, keepdims=True)
    probs = jnp.exp(scores)
    probs = probs / jnp.sum(probs, axis=-1, keepdims=True)

    e0 = route_idx[:, 0]
    e1 = route_idx[:, 1]
    all_ids = jnp.arange(N_EXPERTS, dtype=jnp.int32)
    g0 = jnp.sum(probs * (e0[:, None] == all_ids[None, :]), axis=1)
    g1 = jnp.sum(probs * (e1[:, None] == all_ids[None, :]), axis=1)
    gs = g0 + g1
    w0 = g0 / gs
    w1 = g1 / gs

    local_ids = my * N_LOCAL + jnp.arange(N_LOCAL, dtype=jnp.int32)
    gates = (
        w0[:, None] * (e0[:, None] == local_ids[None, :])
        + w1[:, None] * (e1[:, None] == local_ids[None, :])
    ).astype(jnp.bfloat16)

    def body(
        x_ref,
        g_ref,
        w_ref,
        out_ref,
        xb_ref,
        wb_ref,
        cbuf,
        rs_buf,
        rs_send_sems,
        rs_recv_sems,
        ag_send_sems,
        ag_recv_sems,
    ):
        my_pos = lax.axis_index("i")
        left = (my_pos - 1) % N_DEV
        right = (my_pos + 1) % N_DEV
        nbr_of = [right, left]

        barrier_sem = pltpu.get_barrier_semaphore()
        for nbr in [left, right]:
            pl.semaphore_signal(
                barrier_sem,
                inc=1,
                device_id=(nbr,),
                device_id_type=pl.DeviceIdType.MESH,
            )
        pl.semaphore_wait(barrier_sem, 2)

        xb_ref[:, :] = x_ref[:, :].astype(jnp.bfloat16)
        wb_ref[:, :] = w_ref[:, :].astype(jnp.bfloat16)

        def piece(c, ring):
            rows = pl.ds(c * CHUNK, CHUNK)
            xblk = xb_ref[rows, :]
            gblk = g_ref[rows, :]
            xg_all = jnp.concatenate(
                [xblk * gblk[:, e : e + 1] for e in range(N_LOCAL)], axis=1
            )
            acc = jnp.dot(
                xg_all,
                wb_ref[:, ring * HALF : (ring + 1) * HALF],
                preferred_element_type=jnp.float32,
            )
            cbuf[rows, ring * HALF : (ring + 1) * HALF] = acc.astype(
                jnp.bfloat16
            )

        def mk_rs(ring, h, c_send):
            return pltpu.make_async_remote_copy(
                src_ref=cbuf.at[
                    pl.ds(c_send * CHUNK, CHUNK),
                    pl.ds(ring * HALF, HALF),
                ],
                dst_ref=rs_buf.at[ring, h],
                send_sem=rs_send_sems.at[ring, h],
                recv_sem=rs_recv_sems.at[ring, h],
                device_id=(nbr_of[ring],),
                device_id_type=pl.DeviceIdType.MESH,
            )

        def add_rs(ring, h, c):
            rows = pl.ds(c * CHUNK, CHUNK)
            cols = pl.ds(ring * HALF, HALF)
            cbuf[rows, cols] = cbuf[rows, cols] + rs_buf[ring, h]

        def mk_ag(ring, s, c):
            sl = (pl.ds(c * CHUNK, CHUNK), pl.ds(ring * HALF, HALF))
            return pltpu.make_async_remote_copy(
                src_ref=cbuf.at[sl[0], sl[1]],
                dst_ref=cbuf.at[sl[0], sl[1]],
                send_sem=ag_send_sems.at[ring, s],
                recv_sem=ag_recv_sems.at[ring, s],
                device_id=(nbr_of[ring],),
                device_id_type=pl.DeviceIdType.MESH,
            )

        def conv(c, ring):
            rows = pl.ds(c * CHUNK, CHUNK)
            cols = pl.ds(ring * HALF, HALF)
            out_ref[rows, cols] = cbuf[rows, cols].astype(jnp.float32)

        m = lambda k: (my_pos + k) % N_DEV

        piece(my_pos, 0)
        f0 = mk_rs(0, 0, my_pos)
        f0.start()
        piece(my_pos, 1)
        b0 = mk_rs(1, 0, my_pos)
        b0.start()

        piece(m(-1), 0)
        f0.wait()
        add_rs(0, 0, m(-1))
        f1 = mk_rs(0, 1, m(-1))
        f1.start()

        piece(m(1), 1)
        b0.wait()
        add_rs(1, 0, m(1))
        b1 = mk_rs(1, 1, m(1))
        b1.start()

        piece(m(-2), 0)
        f1.wait()
        add_rs(0, 1, m(-2))
        f2 = mk_rs(0, 2, m(-2))
        f2.start()

        piece(m(2), 1)
        b1.wait()
        add_rs(1, 1, m(2))
        b2 = mk_rs(1, 2, m(2))
        b2.start()

        piece(m(1), 0)

        f2.wait()
        add_rs(0, 2, m(1))

        af0 = mk_ag(0, 0, m(1))
        af0.start()
        piece(m(-1), 1)
        conv(m(1), 0)

        b2.wait()
        add_rs(1, 2, m(-1))
        ab0 = mk_ag(1, 0, m(-1))
        ab0.start()
        conv(m(-1), 1)

        af0.wait()
        af1 = mk_ag(0, 1, m(0))
        af1.start()
        conv(m(0), 0)
        ab0.wait()
        ab1 = mk_ag(1, 1, m(0))
        ab1.start()
        conv(m(0), 1)

        af1.wait()
        af2 = mk_ag(0, 2, m(-1))
        af2.start()
        conv(m(-1), 0)
        ab1.wait()
        ab2 = mk_ag(1, 2, m(1))
        ab2.start()
        conv(m(1), 1)

        af2.wait()
        conv(m(2), 0)
        ab2.wait()
        conv(m(2), 1)

    return pl.pallas_call(
        body,
        out_shape=jax.ShapeDtypeStruct((N_TOK, D_OUT), jnp.float32),
        in_specs=[
            pl.BlockSpec(memory_space=pltpu.VMEM),
            pl.BlockSpec(memory_space=pltpu.VMEM),
            pl.BlockSpec(memory_space=pltpu.VMEM),
        ],
        out_specs=pl.BlockSpec(memory_space=pltpu.VMEM),
        scratch_shapes=[
            pltpu.VMEM((N_TOK, D_IN), jnp.bfloat16),
            pltpu.VMEM((N_LOCAL * D_IN, D_OUT), jnp.bfloat16),
            pltpu.VMEM((N_TOK, D_OUT), jnp.bfloat16),
            pltpu.VMEM((2, N_HOP, CHUNK, HALF), jnp.bfloat16),
            pltpu.SemaphoreType.DMA((2, N_HOP)),
            pltpu.SemaphoreType.DMA((2, N_HOP)),
            pltpu.SemaphoreType.DMA((2, N_HOP)),
            pltpu.SemaphoreType.DMA((2, N_HOP)),
        ],
        compiler_params=pltpu.CompilerParams(
            collective_id=0, vmem_limit_bytes=100 * 1024 * 1024
        ),
    )(x, gates, expert_W.reshape(N_LOCAL * D_IN, D_OUT))
